# baseline (speedup 1.0000x reference)
import functools

import jax
import jax.numpy as jnp
from jax import lax
from jax.experimental import pallas as pl
from jax.experimental.pallas import tpu as pltpu
from jax.experimental.pallas import tpu_sc as plsc

N = 10000
E = 160000
F_IN = 23
H = 8
C = 120
K = 128
DP = H * 128
NP = 10240
NC = 2
NS = 16
NW = NC * NS
EW = E // NW
CH = 40
NCHUNK = EW // CH
ROWS6 = NP // NW

_f32 = jnp.float32

_mesh = plsc.VectorSubcoreMesh(
    core_axis_name="c", subcore_axis_name="s", num_cores=NC, num_subcores=NS
)


def _leaky(t):
    return jnp.maximum(t, 0.2 * t)


_GDN = lax.GatherDimensionNumbers(
    offset_dims=(), collapsed_slice_dims=(0,), start_index_map=(0,)
)


def _vperm(v, idx):
    return lax.gather(
        v, idx[:, None], _GDN, slice_sizes=(1,),
        mode=lax.GatherScatterMode.PROMISE_IN_BOUNDS,
    )


def _lanesum(v, lanes):
    for sft in (8, 4, 2, 1):
        v = v + _vperm(v, lanes ^ sft)
    return v


def _t1_body(x_ref, wl_ref, wr_ref, bl_ref, br_ref, ol_ref, or_ref):
    xb = x_ref[...]
    ol_ref[...] = jnp.dot(xb, wl_ref[...], preferred_element_type=_f32) + bl_ref[...]
    or_ref[...] = jnp.dot(xb, wr_ref[...], preferred_element_type=_f32) + br_ref[...]


def _t1(xp, wl, wr, bl, br):
    return pl.pallas_call(
        _t1_body,
        grid=(NP // 512,),
        in_specs=[
            pl.BlockSpec((512, K), lambda i: (i, 0)),
            pl.BlockSpec((K, DP), lambda i: (0, 0)),
            pl.BlockSpec((K, DP), lambda i: (0, 0)),
            pl.BlockSpec((1, DP), lambda i: (0, 0)),
            pl.BlockSpec((1, DP), lambda i: (0, 0)),
        ],
        out_specs=[
            pl.BlockSpec((512, DP), lambda i: (i, 0)),
            pl.BlockSpec((512, DP), lambda i: (i, 0)),
        ],
        out_shape=[
            jax.ShapeDtypeStruct((NP, DP), _f32),
            jax.ShapeDtypeStruct((NP, DP), _f32),
        ],
    )(xp, wl, wr, bl, br)


HC = 20
NHC = EW // HC

@functools.partial(
    pl.kernel,
    mesh=_mesh,
    compiler_params=pltpu.CompilerParams(use_tc_tiling_on_sc=False),
    out_type=[
        jax.ShapeDtypeStruct((E, 16), _f32),
        jax.ShapeDtypeStruct((NP, 16), _f32),
        jax.ShapeDtypeStruct((NP, 16), _f32),
    ],
    scratch_types=[
        pltpu.VMEM((NHC, HC), jnp.int32),
        pltpu.VMEM((NHC, HC), jnp.int32),
        pltpu.VMEM((HC, DP), _f32),
        pltpu.VMEM((HC, DP), _f32),
        pltpu.VMEM((HC, DP), _f32),
        pltpu.VMEM((HC, DP), _f32),
        pltpu.VMEM((HC, 16), _f32),
        pltpu.VMEM((HC, 16), _f32),
        pltpu.VMEM((H, 128), _f32),
        pltpu.VMEM_SHARED((NP, 16), _f32),
        pltpu.SemaphoreType.DMA,
        pltpu.SemaphoreType.DMA,
        pltpu.SemaphoreType.DMA,
        pltpu.SemaphoreType.DMA,
        pltpu.SemaphoreType.DMA,
        pltpu.SemaphoreType.DMA,
        pltpu.SemaphoreType.DMA,
        pltpu.SemaphoreType.DMA,
    ],
)
def _s1(xl_hbm, xr_hbm, src2_hbm, dst2_hbm, att_hbm, z16_hbm,
        ea_hbm, d0_hbm, d1_hbm,
        srcall, dstall, xl0, xl1, xr0, xr1, er0, er1, attv, dshared,
        gl0, gl1, gr0, gr1, o0, o1, a0, a1):
    c = lax.axis_index("c")
    s = lax.axis_index("s")
    wid = c * NS + s
    base = wid * EW
    pltpu.sync_copy(att_hbm, attv)
    pltpu.sync_copy(src2_hbm.at[wid], srcall)
    pltpu.sync_copy(dst2_hbm.at[wid], dstall)

    @pl.when(s == 0)
    def _():
        pltpu.sync_copy(z16_hbm, dshared)

    plsc.subcore_barrier()
    lanes = lax.iota(jnp.int32, 16)
    maskv = jnp.where(lanes < 8, 1.0, 0.0).astype(_f32)
    onehot = [jnp.where(lanes == h, 1.0, 0.0).astype(_f32) for h in range(H)]

    xlb = (xl0, xl1)
    xrb = (xr0, xr1)
    erb = (er0, er1)
    glb = (gl0, gl1)
    grb = (gr0, gr1)
    ob = (o0, o1)
    ab = (a0, a1)

    def start_gather(n, b):
        pltpu.async_copy(xl_hbm.at[srcall.at[n]], xlb[b], glb[b])
        pltpu.async_copy(xr_hbm.at[dstall.at[n]], xrb[b], grb[b])

    def wait_gather(n, b):
        pltpu.make_async_copy(xl_hbm.at[srcall.at[n]], xlb[b], glb[b]).wait()
        pltpu.make_async_copy(xr_hbm.at[dstall.at[n]], xrb[b], grb[b]).wait()

    def start_out(n, b):
        pltpu.async_copy(erb[b], ea_hbm.at[pl.ds(base + n * HC, HC)], ob[b])
        pltpu.async_copy(erb[b], dshared.at[dstall.at[n]], ab[b], add=True)

    def wait_out(n, b):
        pltpu.make_async_copy(erb[b], ea_hbm.at[pl.ds(base + n * HC, HC)], ob[b]).wait()
        pltpu.make_async_copy(erb[b], dshared.at[dstall.at[n]], ab[b]).wait()

    def compute(n, b):
        xl_, xr_, er_ = xlb[b], xrb[b], erb[b]

        def zero(e, carry2):
            er_[e, :] = jnp.zeros((16,), _f32)
            return carry2

        lax.fori_loop(0, HC, zero, 0, unroll=2)

        for h in range(H):
            atv = [attv[h, pl.ds(j * 16, 16)] for j in range(8)]

            def edge(e, carry2, h=h, atv=atv):
                acc = jnp.zeros((16,), _f32)
                for j in range(8):
                    sl = pl.ds(h * 128 + j * 16, 16)
                    t = xl_[e, sl] + xr_[e, sl]
                    acc = acc + _leaky(t) * atv[j]
                plsc.addupdate(er_.at[e, :], onehot[h] * _lanesum(acc, lanes))
                return carry2

            lax.fori_loop(0, HC, edge, 0, unroll=2)

        def expo(e, carry2):
            er_[e, :] = jnp.exp(er_[e, :]) * maskv
            return carry2

        lax.fori_loop(0, HC, expo, 0, unroll=2)

    start_gather(0, 0)
    start_gather(1, 1)
    wait_gather(0, 0)
    compute(0, 0)
    start_out(0, 0)
    start_gather(2, 0)
    wait_gather(1, 1)
    compute(1, 1)
    start_out(1, 1)
    start_gather(3, 1)

    def pair(p, carry):
        n0 = 2 * p
        n1 = n0 + 1
        wait_gather(n0, 0)
        wait_out(n0 - 2, 0)
        compute(n0, 0)
        start_out(n0, 0)
        start_gather(n0 + 2, 0)
        wait_gather(n1, 1)
        wait_out(n1 - 2, 1)
        compute(n1, 1)
        start_out(n1, 1)
        start_gather(n1 + 2, 1)
        return carry

    lax.fori_loop(1, NHC // 2 - 1, pair, 0)
    nt = NHC - 2
    wait_gather(nt, 0)
    wait_out(nt - 2, 0)
    compute(nt, 0)
    start_out(nt, 0)
    wait_gather(nt + 1, 1)
    wait_out(nt - 1, 1)
    compute(nt + 1, 1)
    start_out(nt + 1, 1)
    wait_out(nt, 0)
    wait_out(nt + 1, 1)

    plsc.subcore_barrier()

    @pl.when((s == 0) & (c == 0))
    def _():
        pltpu.sync_copy(dshared, d0_hbm)

    @pl.when((s == 0) & (c == 1))
    def _():
        pltpu.sync_copy(dshared, d1_hbm)


@functools.partial(
    pl.kernel,
    mesh=_mesh,
    compiler_params=pltpu.CompilerParams(use_tc_tiling_on_sc=False),
    out_type=[jax.ShapeDtypeStruct((E, 16), _f32)],
    scratch_types=[
        pltpu.VMEM((CH,), jnp.int32),
        pltpu.VMEM((CH, 16), _f32),
        pltpu.VMEM((CH, 16), _f32),
        pltpu.VMEM((CH, 16), _f32),
        pltpu.SemaphoreType.DMA,
        pltpu.SemaphoreType.DMA,
    ],
)
def _s2(ea_hbm, dst_hbm, d0_hbm, d1_hbm, coef_hbm,
        dstv, ear, g0, g1, sem1, sem2):
    c = lax.axis_index("c")
    s = lax.axis_index("s")
    base = (c * NS + s) * EW

    def chunk(k, carry):
        off = base + k * CH
        pltpu.sync_copy(dst_hbm.at[pl.ds(off, CH)], dstv)
        pltpu.sync_copy(ea_hbm.at[pl.ds(off, CH)], ear)
        cp1 = pltpu.async_copy(d0_hbm.at[dstv], g0, sem1)
        cp2 = pltpu.async_copy(d1_hbm.at[dstv], g1, sem2)
        cp1.wait()
        cp2.wait()

        def edge(e, carry2):
            dv = g0[e, :] + g1[e, :]
            ear[e, :] = ear[e, :] / (dv + 1e-16)
            return carry2

        lax.fori_loop(0, CH, edge, 0)
        pltpu.sync_copy(ear, coef_hbm.at[pl.ds(off, CH)])
        return carry

    lax.fori_loop(0, NCHUNK, chunk, 0)


@functools.partial(
    pl.kernel,
    mesh=_mesh,
    compiler_params=pltpu.CompilerParams(use_tc_tiling_on_sc=False),
    out_type=[jax.ShapeDtypeStruct((NC * H, NP, 128), _f32)],
    scratch_types=[
        pltpu.VMEM((NCHUNK, CH), jnp.int32),
        pltpu.VMEM((NCHUNK, CH), jnp.int32),
        pltpu.VMEM((CH, 16), _f32),
        pltpu.VMEM((CH, 16), _f32),
        pltpu.VMEM((CH, 128), _f32),
        pltpu.VMEM((CH, 128), _f32),
        pltpu.VMEM((CH, 128), _f32),
        pltpu.VMEM((CH, 128), _f32),
        pltpu.VMEM_SHARED((NP, 128), _f32),
        pltpu.SemaphoreType.DMA,
        pltpu.SemaphoreType.DMA,
        pltpu.SemaphoreType.DMA,
        pltpu.SemaphoreType.DMA,
        pltpu.SemaphoreType.DMA,
        pltpu.SemaphoreType.DMA,
    ],
)
def _s3(xh0, xh1, xh2, xh3, xh4, xh5, xh6, xh7,
        src2_hbm, dst2_hbm, coef_hbm, z128_hbm, msum_hbm,
        srcall, dstall, cb0, cb1, xr0, xr1, msg0, msg1, ashared,
        g0, g1, a0, a1, c0, c1):
    c = lax.axis_index("c")
    s = lax.axis_index("s")
    wid = c * NS + s
    base = wid * EW
    xhs = (xh0, xh1, xh2, xh3, xh4, xh5, xh6, xh7)

    pltpu.sync_copy(src2_hbm.at[wid], srcall)
    pltpu.sync_copy(dst2_hbm.at[wid], dstall)

    xrb = (xr0, xr1)
    msgb = (msg0, msg1)
    cbb = (cb0, cb1)
    gb = (g0, g1)
    ab = (a0, a1)
    cs_ = (c0, c1)

    for h in range(H):
        xlh = xhs[h]

        @pl.when(s == 0)
        def _():
            pltpu.sync_copy(z128_hbm, ashared)

        plsc.subcore_barrier()

        def compute(k, b, h=h):
            xr_, msg_, cb_ = xrb[b], msgb[b], cbb[b]

            def edge(e, carry2):
                cs = cb_[e, :][h]
                for j in range(8):
                    sl = pl.ds(j * 16, 16)
                    msg_[e, sl] = xr_[e, sl] * cs
                return carry2

            lax.fori_loop(0, CH, edge, 0)

        def start_gather(k, b, h=h):
            pltpu.async_copy(coef_hbm.at[pl.ds(base + k * CH, CH)], cbb[b], cs_[b])
            return pltpu.async_copy(xhs[h].at[srcall.at[k]], xrb[b], gb[b])

        def start_scatter(k, b):
            return pltpu.async_copy(msgb[b], ashared.at[dstall.at[k]], ab[b],
                                    add=True)

        def wait_gather(k, b, h=h):
            pltpu.make_async_copy(coef_hbm.at[pl.ds(base + k * CH, CH)], cbb[b], cs_[b]).wait()
            pltpu.make_async_copy(xhs[h].at[srcall.at[k]], xrb[b], gb[b]).wait()

        def wait_scatter(k, b):
            pltpu.make_async_copy(msgb[b], ashared.at[dstall.at[k]], ab[b]).wait()

        start_gather(0, 0)
        start_gather(1, 1)
        wait_gather(0, 0)
        compute(0, 0)
        start_scatter(0, 0)
        start_gather(2, 0)
        wait_gather(1, 1)
        compute(1, 1)
        start_scatter(1, 1)

        def pair(p, carry):
            k0 = 2 * p
            k1 = k0 + 1
            start_gather(k1, 1)
            wait_gather(k0, 0)
            wait_scatter(k0 - 2, 0)
            compute(k0, 0)
            start_scatter(k0, 0)
            start_gather(k1 + 1, 0)
            wait_gather(k1, 1)
            wait_scatter(k1 - 2, 1)
            compute(k1, 1)
            start_scatter(k1, 1)
            return carry

        lax.fori_loop(1, (NCHUNK - 1) // 2, pair, 0)
        kt = NCHUNK - 1
        wait_gather(kt, 0)
        wait_scatter(kt - 2, 0)
        compute(kt, 0)
        start_scatter(kt, 0)
        wait_scatter(kt - 1, 1)
        wait_scatter(kt, 0)

        plsc.subcore_barrier()

        @pl.when(s == 0)
        def _():
            pltpu.sync_copy(ashared, msum_hbm.at[c * H + h])


def _t2_body(m_ref, b1_ref, wl2_ref, wr2_ref, bl2_ref, br2_ref, ol_ref, or_ref):
    m = m_ref[...]
    sgm = m[0:H] + m[H:2 * H]
    h1 = jnp.maximum(sgm + b1_ref[...][:, None, :], 0.0)
    accl = jnp.sum(jnp.sum(h1 * wl2_ref[...][:, None, :], axis=2), axis=0)
    accr = jnp.sum(jnp.sum(h1 * wr2_ref[...][:, None, :], axis=2), axis=0)
    ol_ref[...] = jnp.broadcast_to((accl + bl2_ref[0])[:, None], (512, 128))
    or_ref[...] = jnp.broadcast_to((accr + br2_ref[0])[:, None], (512, 128))


def _t2(msum, b1, wl2, wr2, bl2, br2):
    return pl.pallas_call(
        _t2_body,
        grid=(NP // 512,),
        in_specs=[
            pl.BlockSpec((NC * H, 512, 128), lambda i: (0, i, 0)),
            pl.BlockSpec((H, 128), lambda i: (0, 0)),
            pl.BlockSpec((H, 128), lambda i: (0, 0)),
            pl.BlockSpec((H, 128), lambda i: (0, 0)),
            pl.BlockSpec(memory_space=pltpu.SMEM),
            pl.BlockSpec(memory_space=pltpu.SMEM),
        ],
        out_specs=[
            pl.BlockSpec((512, 128), lambda i: (i, 0)),
            pl.BlockSpec((512, 128), lambda i: (i, 0)),
        ],
        out_shape=[
            jax.ShapeDtypeStruct((NP, 128), _f32),
            jax.ShapeDtypeStruct((NP, 128), _f32),
        ],
    )(msum, b1, wl2, wr2, bl2, br2)


@functools.partial(
    pl.kernel,
    mesh=_mesh,
    compiler_params=pltpu.CompilerParams(use_tc_tiling_on_sc=False),
    out_type=[
        jax.ShapeDtypeStruct((E, 16), _f32),
        jax.ShapeDtypeStruct((NP, 16), _f32),
        jax.ShapeDtypeStruct((NP, 16), _f32),
    ],
    scratch_types=[
        pltpu.VMEM((CH,), jnp.int32),
        pltpu.VMEM((CH,), jnp.int32),
        pltpu.VMEM((CH, 16), _f32),
        pltpu.VMEM((CH, 16), _f32),
        pltpu.VMEM((CH, 16), _f32),
        pltpu.VMEM((16,), _f32),
        pltpu.VMEM_SHARED((NP, 16), _f32),
        pltpu.SemaphoreType.DMA,
        pltpu.SemaphoreType.DMA,
    ],
)
def _s4(xl16, xr16, src_hbm, dst_hbm, att2_hbm, z16_hbm,
        e2_hbm, d0_hbm, d1_hbm,
        srcv, dstv, xs, xd, er, attv, dshared, sem1, sem2):
    c = lax.axis_index("c")
    s = lax.axis_index("s")
    base = (c * NS + s) * EW
    pltpu.sync_copy(att2_hbm, attv)

    @pl.when(s == 0)
    def _():
        pltpu.sync_copy(z16_hbm, dshared)

    plsc.subcore_barrier()
    av = attv[...]

    def chunk(k, carry):
        off = base + k * CH
        pltpu.sync_copy(src_hbm.at[pl.ds(off, CH)], srcv)
        pltpu.sync_copy(dst_hbm.at[pl.ds(off, CH)], dstv)
        cp1 = pltpu.async_copy(xl16.at[srcv], xs, sem1)
        cp2 = pltpu.async_copy(xr16.at[dstv], xd, sem2)
        cp1.wait()
        cp2.wait()

        def edge(e, carry2):
            t = xs[e, :] + xd[e, :]
            er[e, :] = jnp.exp(av * _leaky(t))
            return carry2

        lax.fori_loop(0, CH, edge, 0)
        pltpu.sync_copy(er, e2_hbm.at[pl.ds(off, CH)])
        pltpu.sync_copy(er, dshared.at[dstv], add=True)
        return carry

    lax.fori_loop(0, NCHUNK, chunk, 0)
    plsc.subcore_barrier()

    @pl.when((s == 0) & (c == 0))
    def _():
        pltpu.sync_copy(dshared, d0_hbm)

    @pl.when((s == 0) & (c == 1))
    def _():
        pltpu.sync_copy(dshared, d1_hbm)


@functools.partial(
    pl.kernel,
    mesh=_mesh,
    compiler_params=pltpu.CompilerParams(use_tc_tiling_on_sc=False),
    out_type=[
        jax.ShapeDtypeStruct((NP, 16), _f32),
        jax.ShapeDtypeStruct((NP, 16), _f32),
    ],
    scratch_types=[
        pltpu.VMEM((CH,), jnp.int32),
        pltpu.VMEM((CH,), jnp.int32),
        pltpu.VMEM((CH, 16), _f32),
        pltpu.VMEM((CH, 16), _f32),
        pltpu.VMEM((CH, 16), _f32),
        pltpu.VMEM((CH, 16), _f32),
        pltpu.VMEM_SHARED((NP, 16), _f32),
        pltpu.SemaphoreType.DMA,
        pltpu.SemaphoreType.DMA,
        pltpu.SemaphoreType.DMA,
    ],
)
def _s5(xl16, src_hbm, dst_hbm, e2_hbm, d20_hbm, d21_hbm, z16_hbm,
        o0_hbm, o1_hbm,
        srcv, dstv, xs, er, g0, g1, oshared, sem1, sem2, sem3):
    c = lax.axis_index("c")
    s = lax.axis_index("s")
    base = (c * NS + s) * EW

    @pl.when(s == 0)
    def _():
        pltpu.sync_copy(z16_hbm, oshared)

    plsc.subcore_barrier()

    def chunk(k, carry):
        off = base + k * CH
        pltpu.sync_copy(src_hbm.at[pl.ds(off, CH)], srcv)
        pltpu.sync_copy(dst_hbm.at[pl.ds(off, CH)], dstv)
        pltpu.sync_copy(e2_hbm.at[pl.ds(off, CH)], er)
        cp1 = pltpu.async_copy(xl16.at[srcv], xs, sem1)
        cp2 = pltpu.async_copy(d20_hbm.at[dstv], g0, sem2)
        cp3 = pltpu.async_copy(d21_hbm.at[dstv], g1, sem3)
        cp1.wait()
        cp2.wait()
        cp3.wait()

        def edge(e, carry2):
            dv = g0[e, :] + g1[e, :]
            cv = er[e, :] / (dv + 1e-16)
            er[e, :] = xs[e, :] * cv
            return carry2

        lax.fori_loop(0, CH, edge, 0)
        pltpu.sync_copy(er, oshared.at[dstv], add=True)
        return carry

    lax.fori_loop(0, NCHUNK, chunk, 0)
    plsc.subcore_barrier()

    @pl.when((s == 0) & (c == 0))
    def _():
        pltpu.sync_copy(oshared, o0_hbm)

    @pl.when((s == 0) & (c == 1))
    def _():
        pltpu.sync_copy(oshared, o1_hbm)


@functools.partial(
    pl.kernel,
    mesh=_mesh,
    compiler_params=pltpu.CompilerParams(use_tc_tiling_on_sc=False),
    out_type=[jax.ShapeDtypeStruct((NP, 16), _f32)],
    scratch_types=[
        pltpu.VMEM((ROWS6, 16), _f32),
        pltpu.VMEM((ROWS6, 16), _f32),
        pltpu.VMEM((16,), _f32),
    ],
)
def _s6(o0_hbm, o1_hbm, b2_hbm, out_hbm, p0, p1, bv):
    c = lax.axis_index("c")
    s = lax.axis_index("s")
    base = (c * NS + s) * ROWS6
    pltpu.sync_copy(o0_hbm.at[pl.ds(base, ROWS6)], p0)
    pltpu.sync_copy(o1_hbm.at[pl.ds(base, ROWS6)], p1)
    pltpu.sync_copy(b2_hbm, bv)
    bvv = bv[...]

    def row(r, carry):
        p0[r, :] = p0[r, :] + p1[r, :] + bvv
        return carry

    lax.fori_loop(0, ROWS6, row, 0)
    pltpu.sync_copy(p0, out_hbm.at[pl.ds(base, ROWS6)])


def kernel(x, edge_index, Wl1, bl1, Wr1, br1, att1, bias1,
           Wl2, bl2, Wr2, br2, att2, bias2):
    src = edge_index[0]
    dst = edge_index[1]

    xp = jnp.zeros((NP, K), _f32).at[:N, :F_IN].set(x)

    def padw1(W, b):
        Wp = jnp.zeros((K, H, 128), _f32).at[:F_IN, :, :C].set(W.reshape(F_IN, H, C))
        bp = jnp.zeros((H, 128), _f32).at[:, :C].set(b.reshape(H, C))
        return Wp.reshape(K, DP), bp.reshape(1, DP)

    Wl1p, bl1p = padw1(Wl1, bl1)
    Wr1p, br1p = padw1(Wr1, br1)
    att1p = jnp.zeros((H, 128), _f32).at[:, :C].set(att1.reshape(H, C))
    bias1p = jnp.zeros((H, 128), _f32).at[:, :C].set(bias1.reshape(H, C))
    Wl2p = jnp.zeros((H, 128), _f32).at[:, :C].set(Wl2.reshape(H, C))
    Wr2p = jnp.zeros((H, 128), _f32).at[:, :C].set(Wr2.reshape(H, C))
    att2v = jnp.full((16,), att2.reshape(()), _f32)
    b2v = jnp.full((16,), bias2.reshape(()), _f32)
    z16 = jnp.zeros((NP, 16), _f32)
    z128 = jnp.zeros((NP, 128), _f32)

    xl1p, xr1p = _t1(xp, Wl1p, Wr1p, bl1p, br1p)

    srch = src.reshape(NW, NHC, HC)
    dsth = dst.reshape(NW, NHC, HC)
    ea, d0, d1 = _s1(xl1p, xr1p, srch, dsth, att1p, z16)
    (coef,) = _s2(ea, dst, d0, d1)
    xh = xl1p.reshape(NP, H, 128)
    xhs = [xh[:, h, :] for h in range(H)]
    src2 = src.reshape(NW, NCHUNK, CH)
    dst2 = dst.reshape(NW, NCHUNK, CH)
    (msum,) = _s3(*xhs, src2, dst2, coef, z128)

    xl2b, xr2b = _t2(msum, bias1p, Wl2p, Wr2p, bl2, br2)
    xl2_16 = xl2b[:, :16]
    xr2_16 = xr2b[:, :16]

    e2, d20, d21 = _s4(xl2_16, xr2_16, src, dst, att2v, z16)
    o0, o1 = _s5(xl2_16, src, dst, e2, d20, d21, z16)
    (out16,) = _s6(o0, o1, b2v)

    return out16[:N, :1]


# speedup vs baseline: 1.1961x; 1.1961x over previous
import functools

import jax
import jax.numpy as jnp
from jax import lax
from jax.experimental import pallas as pl
from jax.experimental.pallas import tpu as pltpu
from jax.experimental.pallas import tpu_sc as plsc

N = 10000
E = 160000
F_IN = 23
H = 8
C = 120
K = 128
DP = H * 128
NP = 10240
NC = 2
NS = 16
NW = NC * NS
EW = E // NW
CH = 40
NCHUNK = EW // CH
ROWS6 = NP // NW

_f32 = jnp.float32

_mesh = plsc.VectorSubcoreMesh(
    core_axis_name="c", subcore_axis_name="s", num_cores=NC, num_subcores=NS
)


def _leaky(t):
    return jnp.maximum(t, 0.2 * t)


_GDN = lax.GatherDimensionNumbers(
    offset_dims=(), collapsed_slice_dims=(0,), start_index_map=(0,)
)


def _vperm(v, idx):
    return lax.gather(
        v, idx[:, None], _GDN, slice_sizes=(1,),
        mode=lax.GatherScatterMode.PROMISE_IN_BOUNDS,
    )


def _lanesum(v, lanes):
    for sft in (8, 4, 2, 1):
        v = v + _vperm(v, lanes ^ sft)
    return v


def _t1_body(x_ref, wl_ref, wr_ref, bl_ref, br_ref, ol_ref, or_ref):
    xb = x_ref[...]
    ol_ref[...] = jnp.dot(xb, wl_ref[...], preferred_element_type=_f32) + bl_ref[...]
    or_ref[...] = jnp.dot(xb, wr_ref[...], preferred_element_type=_f32) + br_ref[...]


def _t1(xp, wl, wr, bl, br):
    return pl.pallas_call(
        _t1_body,
        grid=(NP // 512,),
        in_specs=[
            pl.BlockSpec((512, K), lambda i: (i, 0)),
            pl.BlockSpec((K, DP), lambda i: (0, 0)),
            pl.BlockSpec((K, DP), lambda i: (0, 0)),
            pl.BlockSpec((1, DP), lambda i: (0, 0)),
            pl.BlockSpec((1, DP), lambda i: (0, 0)),
        ],
        out_specs=[
            pl.BlockSpec((512, DP), lambda i: (i, 0)),
            pl.BlockSpec((512, DP), lambda i: (i, 0)),
        ],
        out_shape=[
            jax.ShapeDtypeStruct((NP, DP), _f32),
            jax.ShapeDtypeStruct((NP, DP), _f32),
        ],
    )(xp, wl, wr, bl, br)


HC = 20
NHC = EW // HC

@functools.partial(
    pl.kernel,
    mesh=_mesh,
    compiler_params=pltpu.CompilerParams(use_tc_tiling_on_sc=False),
    out_type=[
        jax.ShapeDtypeStruct((E, 16), _f32),
        jax.ShapeDtypeStruct((NP, 16), _f32),
        jax.ShapeDtypeStruct((NP, 16), _f32),
    ],
    scratch_types=[
        pltpu.VMEM((NHC, HC), jnp.int32),
        pltpu.VMEM((NHC, HC), jnp.int32),
        pltpu.VMEM((HC, DP), _f32),
        pltpu.VMEM((HC, DP), _f32),
        pltpu.VMEM((HC, DP), _f32),
        pltpu.VMEM((HC, DP), _f32),
        pltpu.VMEM((HC, 16), _f32),
        pltpu.VMEM((HC, 16), _f32),
        pltpu.VMEM((H, 128), _f32),
        pltpu.VMEM_SHARED((NP, 16), _f32),
        pltpu.SemaphoreType.DMA,
        pltpu.SemaphoreType.DMA,
        pltpu.SemaphoreType.DMA,
        pltpu.SemaphoreType.DMA,
        pltpu.SemaphoreType.DMA,
        pltpu.SemaphoreType.DMA,
        pltpu.SemaphoreType.DMA,
        pltpu.SemaphoreType.DMA,
    ],
)
def _s1(xl_hbm, xr_hbm, src2_hbm, dst2_hbm, att_hbm, z16_hbm,
        ea_hbm, d0_hbm, d1_hbm,
        srcall, dstall, xl0, xl1, xr0, xr1, er0, er1, attv, dshared,
        gl0, gl1, gr0, gr1, o0, o1, a0, a1):
    c = lax.axis_index("c")
    s = lax.axis_index("s")
    wid = c * NS + s
    base = wid * EW
    pltpu.sync_copy(att_hbm, attv)
    pltpu.sync_copy(src2_hbm.at[wid], srcall)
    pltpu.sync_copy(dst2_hbm.at[wid], dstall)

    @pl.when(s == 0)
    def _():
        pltpu.sync_copy(z16_hbm, dshared)

    plsc.subcore_barrier()
    lanes = lax.iota(jnp.int32, 16)
    maskv = jnp.where(lanes < 8, 1.0, 0.0).astype(_f32)
    onehot = [jnp.where(lanes == h, 1.0, 0.0).astype(_f32) for h in range(H)]

    xlb = (xl0, xl1)
    xrb = (xr0, xr1)
    erb = (er0, er1)
    glb = (gl0, gl1)
    grb = (gr0, gr1)
    ob = (o0, o1)
    ab = (a0, a1)

    def start_gather(n, b):
        pltpu.async_copy(xl_hbm.at[srcall.at[n]], xlb[b], glb[b])
        pltpu.async_copy(xr_hbm.at[dstall.at[n]], xrb[b], grb[b])

    def wait_gather(n, b):
        pltpu.make_async_copy(xl_hbm.at[srcall.at[n]], xlb[b], glb[b]).wait()
        pltpu.make_async_copy(xr_hbm.at[dstall.at[n]], xrb[b], grb[b]).wait()

    def start_out(n, b):
        pltpu.async_copy(erb[b], ea_hbm.at[pl.ds(base + n * HC, HC)], ob[b])
        pltpu.async_copy(erb[b], dshared.at[dstall.at[n]], ab[b], add=True)

    def wait_out(n, b):
        pltpu.make_async_copy(erb[b], ea_hbm.at[pl.ds(base + n * HC, HC)], ob[b]).wait()
        pltpu.make_async_copy(erb[b], dshared.at[dstall.at[n]], ab[b]).wait()

    def compute(n, b):
        xl_, xr_, er_ = xlb[b], xrb[b], erb[b]

        def edge(e, carry2):
            alphav = jnp.zeros((16,), _f32)
            for h in range(H):
                acc = jnp.zeros((16,), _f32)
                for j in range(8):
                    sl = pl.ds(h * 128 + j * 16, 16)
                    t = xl_[e, sl] + xr_[e, sl]
                    acc = acc + _leaky(t) * attv[h, pl.ds(j * 16, 16)]
                alphav = alphav + onehot[h] * _lanesum(acc, lanes)
            er_[e, :] = jnp.exp(alphav) * maskv
            return carry2

        lax.fori_loop(0, HC, edge, 0, unroll=2)

    start_gather(0, 0)
    start_gather(1, 1)
    wait_gather(0, 0)
    compute(0, 0)
    start_out(0, 0)
    start_gather(2, 0)
    wait_gather(1, 1)
    compute(1, 1)
    start_out(1, 1)
    start_gather(3, 1)

    def pair(p, carry):
        n0 = 2 * p
        n1 = n0 + 1
        wait_gather(n0, 0)
        wait_out(n0 - 2, 0)
        compute(n0, 0)
        start_out(n0, 0)
        start_gather(n0 + 2, 0)
        wait_gather(n1, 1)
        wait_out(n1 - 2, 1)
        compute(n1, 1)
        start_out(n1, 1)
        start_gather(n1 + 2, 1)
        return carry

    lax.fori_loop(1, NHC // 2 - 1, pair, 0)
    nt = NHC - 2
    wait_gather(nt, 0)
    wait_out(nt - 2, 0)
    compute(nt, 0)
    start_out(nt, 0)
    wait_gather(nt + 1, 1)
    wait_out(nt - 1, 1)
    compute(nt + 1, 1)
    start_out(nt + 1, 1)
    wait_out(nt, 0)
    wait_out(nt + 1, 1)

    plsc.subcore_barrier()

    @pl.when((s == 0) & (c == 0))
    def _():
        pltpu.sync_copy(dshared, d0_hbm)

    @pl.when((s == 0) & (c == 1))
    def _():
        pltpu.sync_copy(dshared, d1_hbm)


@functools.partial(
    pl.kernel,
    mesh=_mesh,
    compiler_params=pltpu.CompilerParams(use_tc_tiling_on_sc=False),
    out_type=[jax.ShapeDtypeStruct((E, 16), _f32)],
    scratch_types=[
        pltpu.VMEM((CH,), jnp.int32),
        pltpu.VMEM((CH, 16), _f32),
        pltpu.VMEM((CH, 16), _f32),
        pltpu.VMEM((CH, 16), _f32),
        pltpu.SemaphoreType.DMA,
        pltpu.SemaphoreType.DMA,
    ],
)
def _s2(ea_hbm, dst_hbm, d0_hbm, d1_hbm, coef_hbm,
        dstv, ear, g0, g1, sem1, sem2):
    c = lax.axis_index("c")
    s = lax.axis_index("s")
    base = (c * NS + s) * EW

    def chunk(k, carry):
        off = base + k * CH
        pltpu.sync_copy(dst_hbm.at[pl.ds(off, CH)], dstv)
        pltpu.sync_copy(ea_hbm.at[pl.ds(off, CH)], ear)
        cp1 = pltpu.async_copy(d0_hbm.at[dstv], g0, sem1)
        cp2 = pltpu.async_copy(d1_hbm.at[dstv], g1, sem2)
        cp1.wait()
        cp2.wait()

        def edge(e, carry2):
            dv = g0[e, :] + g1[e, :]
            ear[e, :] = ear[e, :] / (dv + 1e-16)
            return carry2

        lax.fori_loop(0, CH, edge, 0)
        pltpu.sync_copy(ear, coef_hbm.at[pl.ds(off, CH)])
        return carry

    lax.fori_loop(0, NCHUNK, chunk, 0)


@functools.partial(
    pl.kernel,
    mesh=_mesh,
    compiler_params=pltpu.CompilerParams(use_tc_tiling_on_sc=False),
    out_type=[jax.ShapeDtypeStruct((NC * H, NP, 128), _f32)],
    scratch_types=[
        pltpu.VMEM((NCHUNK, CH), jnp.int32),
        pltpu.VMEM((NCHUNK, CH), jnp.int32),
        pltpu.VMEM((CH, 16), _f32),
        pltpu.VMEM((CH, 16), _f32),
        pltpu.VMEM((CH, 128), _f32),
        pltpu.VMEM((CH, 128), _f32),
        pltpu.VMEM((CH, 128), _f32),
        pltpu.VMEM((CH, 128), _f32),
        pltpu.VMEM_SHARED((NP, 128), _f32),
        pltpu.SemaphoreType.DMA,
        pltpu.SemaphoreType.DMA,
        pltpu.SemaphoreType.DMA,
        pltpu.SemaphoreType.DMA,
        pltpu.SemaphoreType.DMA,
        pltpu.SemaphoreType.DMA,
    ],
)
def _s3(xh0, xh1, xh2, xh3, xh4, xh5, xh6, xh7,
        src2_hbm, dst2_hbm, coef_hbm, z128_hbm, msum_hbm,
        srcall, dstall, cb0, cb1, xr0, xr1, msg0, msg1, ashared,
        g0, g1, a0, a1, c0, c1):
    c = lax.axis_index("c")
    s = lax.axis_index("s")
    wid = c * NS + s
    base = wid * EW
    xhs = (xh0, xh1, xh2, xh3, xh4, xh5, xh6, xh7)

    pltpu.sync_copy(src2_hbm.at[wid], srcall)
    pltpu.sync_copy(dst2_hbm.at[wid], dstall)

    xrb = (xr0, xr1)
    msgb = (msg0, msg1)
    cbb = (cb0, cb1)
    gb = (g0, g1)
    ab = (a0, a1)
    cs_ = (c0, c1)

    for h in range(H):
        xlh = xhs[h]

        @pl.when(s == 0)
        def _():
            pltpu.sync_copy(z128_hbm, ashared)

        plsc.subcore_barrier()

        def compute(k, b, h=h):
            xr_, msg_, cb_ = xrb[b], msgb[b], cbb[b]

            def edge(e, carry2):
                cs = cb_[e, :][h]
                for j in range(8):
                    sl = pl.ds(j * 16, 16)
                    msg_[e, sl] = xr_[e, sl] * cs
                return carry2

            lax.fori_loop(0, CH, edge, 0)

        def start_gather(k, b, h=h):
            pltpu.async_copy(coef_hbm.at[pl.ds(base + k * CH, CH)], cbb[b], cs_[b])
            return pltpu.async_copy(xhs[h].at[srcall.at[k]], xrb[b], gb[b])

        def start_scatter(k, b):
            return pltpu.async_copy(msgb[b], ashared.at[dstall.at[k]], ab[b],
                                    add=True)

        def wait_gather(k, b, h=h):
            pltpu.make_async_copy(coef_hbm.at[pl.ds(base + k * CH, CH)], cbb[b], cs_[b]).wait()
            pltpu.make_async_copy(xhs[h].at[srcall.at[k]], xrb[b], gb[b]).wait()

        def wait_scatter(k, b):
            pltpu.make_async_copy(msgb[b], ashared.at[dstall.at[k]], ab[b]).wait()

        start_gather(0, 0)
        start_gather(1, 1)
        wait_gather(0, 0)
        compute(0, 0)
        start_scatter(0, 0)
        start_gather(2, 0)
        wait_gather(1, 1)
        compute(1, 1)
        start_scatter(1, 1)

        def pair(p, carry):
            k0 = 2 * p
            k1 = k0 + 1
            start_gather(k1, 1)
            wait_gather(k0, 0)
            wait_scatter(k0 - 2, 0)
            compute(k0, 0)
            start_scatter(k0, 0)
            start_gather(k1 + 1, 0)
            wait_gather(k1, 1)
            wait_scatter(k1 - 2, 1)
            compute(k1, 1)
            start_scatter(k1, 1)
            return carry

        lax.fori_loop(1, (NCHUNK - 1) // 2, pair, 0)
        kt = NCHUNK - 1
        wait_gather(kt, 0)
        wait_scatter(kt - 2, 0)
        compute(kt, 0)
        start_scatter(kt, 0)
        wait_scatter(kt - 1, 1)
        wait_scatter(kt, 0)

        plsc.subcore_barrier()

        @pl.when(s == 0)
        def _():
            pltpu.sync_copy(ashared, msum_hbm.at[c * H + h])


def _t2_body(m_ref, b1_ref, wl2_ref, wr2_ref, bl2_ref, br2_ref, ol_ref, or_ref):
    m = m_ref[...]
    sgm = m[0:H] + m[H:2 * H]
    h1 = jnp.maximum(sgm + b1_ref[...][:, None, :], 0.0)
    accl = jnp.sum(jnp.sum(h1 * wl2_ref[...][:, None, :], axis=2), axis=0)
    accr = jnp.sum(jnp.sum(h1 * wr2_ref[...][:, None, :], axis=2), axis=0)
    ol_ref[...] = jnp.broadcast_to((accl + bl2_ref[0])[:, None], (512, 128))
    or_ref[...] = jnp.broadcast_to((accr + br2_ref[0])[:, None], (512, 128))


def _t2(msum, b1, wl2, wr2, bl2, br2):
    return pl.pallas_call(
        _t2_body,
        grid=(NP // 512,),
        in_specs=[
            pl.BlockSpec((NC * H, 512, 128), lambda i: (0, i, 0)),
            pl.BlockSpec((H, 128), lambda i: (0, 0)),
            pl.BlockSpec((H, 128), lambda i: (0, 0)),
            pl.BlockSpec((H, 128), lambda i: (0, 0)),
            pl.BlockSpec(memory_space=pltpu.SMEM),
            pl.BlockSpec(memory_space=pltpu.SMEM),
        ],
        out_specs=[
            pl.BlockSpec((512, 128), lambda i: (i, 0)),
            pl.BlockSpec((512, 128), lambda i: (i, 0)),
        ],
        out_shape=[
            jax.ShapeDtypeStruct((NP, 128), _f32),
            jax.ShapeDtypeStruct((NP, 128), _f32),
        ],
    )(msum, b1, wl2, wr2, bl2, br2)


@functools.partial(
    pl.kernel,
    mesh=_mesh,
    compiler_params=pltpu.CompilerParams(use_tc_tiling_on_sc=False),
    out_type=[
        jax.ShapeDtypeStruct((E, 16), _f32),
        jax.ShapeDtypeStruct((NP, 16), _f32),
        jax.ShapeDtypeStruct((NP, 16), _f32),
    ],
    scratch_types=[
        pltpu.VMEM((CH,), jnp.int32),
        pltpu.VMEM((CH,), jnp.int32),
        pltpu.VMEM((CH, 16), _f32),
        pltpu.VMEM((CH, 16), _f32),
        pltpu.VMEM((CH, 16), _f32),
        pltpu.VMEM((16,), _f32),
        pltpu.VMEM_SHARED((NP, 16), _f32),
        pltpu.SemaphoreType.DMA,
        pltpu.SemaphoreType.DMA,
    ],
)
def _s4(xl16, xr16, src_hbm, dst_hbm, att2_hbm, z16_hbm,
        e2_hbm, d0_hbm, d1_hbm,
        srcv, dstv, xs, xd, er, attv, dshared, sem1, sem2):
    c = lax.axis_index("c")
    s = lax.axis_index("s")
    base = (c * NS + s) * EW
    pltpu.sync_copy(att2_hbm, attv)

    @pl.when(s == 0)
    def _():
        pltpu.sync_copy(z16_hbm, dshared)

    plsc.subcore_barrier()
    av = attv[...]

    def chunk(k, carry):
        off = base + k * CH
        pltpu.sync_copy(src_hbm.at[pl.ds(off, CH)], srcv)
        pltpu.sync_copy(dst_hbm.at[pl.ds(off, CH)], dstv)
        cp1 = pltpu.async_copy(xl16.at[srcv], xs, sem1)
        cp2 = pltpu.async_copy(xr16.at[dstv], xd, sem2)
        cp1.wait()
        cp2.wait()

        def edge(e, carry2):
            t = xs[e, :] + xd[e, :]
            er[e, :] = jnp.exp(av * _leaky(t))
            return carry2

        lax.fori_loop(0, CH, edge, 0)
        pltpu.sync_copy(er, e2_hbm.at[pl.ds(off, CH)])
        pltpu.sync_copy(er, dshared.at[dstv], add=True)
        return carry

    lax.fori_loop(0, NCHUNK, chunk, 0)
    plsc.subcore_barrier()

    @pl.when((s == 0) & (c == 0))
    def _():
        pltpu.sync_copy(dshared, d0_hbm)

    @pl.when((s == 0) & (c == 1))
    def _():
        pltpu.sync_copy(dshared, d1_hbm)


@functools.partial(
    pl.kernel,
    mesh=_mesh,
    compiler_params=pltpu.CompilerParams(use_tc_tiling_on_sc=False),
    out_type=[
        jax.ShapeDtypeStruct((NP, 16), _f32),
        jax.ShapeDtypeStruct((NP, 16), _f32),
    ],
    scratch_types=[
        pltpu.VMEM((CH,), jnp.int32),
        pltpu.VMEM((CH,), jnp.int32),
        pltpu.VMEM((CH, 16), _f32),
        pltpu.VMEM((CH, 16), _f32),
        pltpu.VMEM((CH, 16), _f32),
        pltpu.VMEM((CH, 16), _f32),
        pltpu.VMEM_SHARED((NP, 16), _f32),
        pltpu.SemaphoreType.DMA,
        pltpu.SemaphoreType.DMA,
        pltpu.SemaphoreType.DMA,
    ],
)
def _s5(xl16, src_hbm, dst_hbm, e2_hbm, d20_hbm, d21_hbm, z16_hbm,
        o0_hbm, o1_hbm,
        srcv, dstv, xs, er, g0, g1, oshared, sem1, sem2, sem3):
    c = lax.axis_index("c")
    s = lax.axis_index("s")
    base = (c * NS + s) * EW

    @pl.when(s == 0)
    def _():
        pltpu.sync_copy(z16_hbm, oshared)

    plsc.subcore_barrier()

    def chunk(k, carry):
        off = base + k * CH
        pltpu.sync_copy(src_hbm.at[pl.ds(off, CH)], srcv)
        pltpu.sync_copy(dst_hbm.at[pl.ds(off, CH)], dstv)
        pltpu.sync_copy(e2_hbm.at[pl.ds(off, CH)], er)
        cp1 = pltpu.async_copy(xl16.at[srcv], xs, sem1)
        cp2 = pltpu.async_copy(d20_hbm.at[dstv], g0, sem2)
        cp3 = pltpu.async_copy(d21_hbm.at[dstv], g1, sem3)
        cp1.wait()
        cp2.wait()
        cp3.wait()

        def edge(e, carry2):
            dv = g0[e, :] + g1[e, :]
            cv = er[e, :] / (dv + 1e-16)
            er[e, :] = xs[e, :] * cv
            return carry2

        lax.fori_loop(0, CH, edge, 0)
        pltpu.sync_copy(er, oshared.at[dstv], add=True)
        return carry

    lax.fori_loop(0, NCHUNK, chunk, 0)
    plsc.subcore_barrier()

    @pl.when((s == 0) & (c == 0))
    def _():
        pltpu.sync_copy(oshared, o0_hbm)

    @pl.when((s == 0) & (c == 1))
    def _():
        pltpu.sync_copy(oshared, o1_hbm)


@functools.partial(
    pl.kernel,
    mesh=_mesh,
    compiler_params=pltpu.CompilerParams(use_tc_tiling_on_sc=False),
    out_type=[jax.ShapeDtypeStruct((NP, 16), _f32)],
    scratch_types=[
        pltpu.VMEM((ROWS6, 16), _f32),
        pltpu.VMEM((ROWS6, 16), _f32),
        pltpu.VMEM((16,), _f32),
    ],
)
def _s6(o0_hbm, o1_hbm, b2_hbm, out_hbm, p0, p1, bv):
    c = lax.axis_index("c")
    s = lax.axis_index("s")
    base = (c * NS + s) * ROWS6
    pltpu.sync_copy(o0_hbm.at[pl.ds(base, ROWS6)], p0)
    pltpu.sync_copy(o1_hbm.at[pl.ds(base, ROWS6)], p1)
    pltpu.sync_copy(b2_hbm, bv)
    bvv = bv[...]

    def row(r, carry):
        p0[r, :] = p0[r, :] + p1[r, :] + bvv
        return carry

    lax.fori_loop(0, ROWS6, row, 0)
    pltpu.sync_copy(p0, out_hbm.at[pl.ds(base, ROWS6)])


def kernel(x, edge_index, Wl1, bl1, Wr1, br1, att1, bias1,
           Wl2, bl2, Wr2, br2, att2, bias2):
    src = edge_index[0]
    dst = edge_index[1]

    xp = jnp.zeros((NP, K), _f32).at[:N, :F_IN].set(x)

    def padw1(W, b):
        Wp = jnp.zeros((K, H, 128), _f32).at[:F_IN, :, :C].set(W.reshape(F_IN, H, C))
        bp = jnp.zeros((H, 128), _f32).at[:, :C].set(b.reshape(H, C))
        return Wp.reshape(K, DP), bp.reshape(1, DP)

    Wl1p, bl1p = padw1(Wl1, bl1)
    Wr1p, br1p = padw1(Wr1, br1)
    att1p = jnp.zeros((H, 128), _f32).at[:, :C].set(att1.reshape(H, C))
    bias1p = jnp.zeros((H, 128), _f32).at[:, :C].set(bias1.reshape(H, C))
    Wl2p = jnp.zeros((H, 128), _f32).at[:, :C].set(Wl2.reshape(H, C))
    Wr2p = jnp.zeros((H, 128), _f32).at[:, :C].set(Wr2.reshape(H, C))
    att2v = jnp.full((16,), att2.reshape(()), _f32)
    b2v = jnp.full((16,), bias2.reshape(()), _f32)
    z16 = jnp.zeros((NP, 16), _f32)
    z128 = jnp.zeros((NP, 128), _f32)

    xl1p, xr1p = _t1(xp, Wl1p, Wr1p, bl1p, br1p)

    srch = src.reshape(NW, NHC, HC)
    dsth = dst.reshape(NW, NHC, HC)
    ea, d0, d1 = _s1(xl1p, xr1p, srch, dsth, att1p, z16)
    (coef,) = _s2(ea, dst, d0, d1)
    xh = xl1p.reshape(NP, H, 128)
    xhs = [xh[:, h, :] for h in range(H)]
    src2 = src.reshape(NW, NCHUNK, CH)
    dst2 = dst.reshape(NW, NCHUNK, CH)
    (msum,) = _s3(*xhs, src2, dst2, coef, z128)

    xl2b, xr2b = _t2(msum, bias1p, Wl2p, Wr2p, bl2, br2)
    xl2_16 = xl2b[:, :16]
    xr2_16 = xr2b[:, :16]

    e2, d20, d21 = _s4(xl2_16, xr2_16, src, dst, att2v, z16)
    o0, o1 = _s5(xl2_16, src, dst, e2, d20, d21, z16)
    (out16,) = _s6(o0, o1, b2v)

    return out16[:N, :1]
